# Initial kernel scaffold; baseline (speedup 1.0000x reference)
import jax
import jax.numpy as jnp
from jax.experimental import pallas as pl


def kernel(data, table):
    raise NotImplementedError("write your pallas kernel here")


# speedup vs baseline: 3.3727x; 3.3727x over previous
import functools

import jax
import jax.numpy as jnp
from jax import lax
from jax.experimental import pallas as pl
from jax.experimental.pallas import tpu as pltpu
from jax.experimental.pallas import tpu_sc as plsc

FIELD_SIZE = 3847
NUM_FIELDS = 26
NC = 2
NS = 16
NW = NC * NS
GROUP = 128
NBUF = 4


def _make_kernel(N, V, D):
    b_per_w = N // NW
    n_groups = b_per_w // GROUP
    mesh = plsc.VectorSubcoreMesh(core_axis_name="c", subcore_axis_name="s")

    @functools.partial(
        pl.kernel,
        mesh=mesh,
        out_type=jax.ShapeDtypeStruct((N, D), jnp.float32),
        scratch_types=[
            pltpu.VMEM((n_groups, GROUP), jnp.int32),
            *[pltpu.VMEM((GROUP, D), jnp.float32) for _ in range(NBUF)],
            *[pltpu.SemaphoreType.DMA for _ in range(NBUF)],
            *[pltpu.SemaphoreType.DMA for _ in range(NBUF)],
        ],
    )
    def embed(data_hbm, table_hbm, out_hbm, idx_v, *rest):
        bufs = rest[:NBUF]
        gsems = rest[NBUF:2 * NBUF]
        ssems = rest[2 * NBUF:]

        wid = lax.axis_index("s") * NC + lax.axis_index("c")
        grp0 = wid * n_groups
        row0 = wid * b_per_w
        lanes = lax.iota(jnp.int32, 16)

        pltpu.sync_copy(data_hbm.at[pl.ds(grp0, n_groups)], idx_v)

        def add_offsets(r):
            p_row = (grp0 + r) * GROUP
            for c in range(GROUP // 16):
                p = p_row + c * 16 + lanes
                off = lax.rem(p, NUM_FIELDS) * FIELD_SIZE
                sl = pl.ds(c * 16, 16)
                idx_v[r, sl] = idx_v[r, sl] + off

        def start_gather(g, b):
            pltpu.async_copy(table_hbm.at[idx_v.at[g]], bufs[b], gsems[b])

        def wait_gather(g, b):
            pltpu.make_async_copy(table_hbm.at[idx_v.at[g]], bufs[b],
                                  gsems[b]).wait()

        def out_slice(g):
            return out_hbm.at[pl.ds(row0 + g * GROUP, GROUP)]

        for b in range(NBUF):
            add_offsets(b)
            start_gather(b, b)

        def step(o, _):
            for b in range(NBUF):
                g = o * NBUF + b
                wait_gather(g, b)
                pltpu.async_copy(bufs[b], out_slice(g), ssems[b])
                gn = g + NBUF

                @pl.when(gn < n_groups)
                def _():
                    pltpu.make_async_copy(bufs[b], out_slice(g),
                                          ssems[b]).wait()
                    add_offsets(gn)
                    start_gather(gn, b)

            return None

        lax.fori_loop(0, n_groups // NBUF, step, None)

        for b in range(NBUF):
            g = n_groups - NBUF + b
            pltpu.make_async_copy(bufs[b], out_slice(g), ssems[b]).wait()

    return embed


def kernel(data, table):
    B, F = data.shape
    V, D = table.shape
    N = B * F
    dataf = data.reshape(N // GROUP, GROUP).astype(jnp.int32)
    embed = _make_kernel(N, V, D)
    out = embed(dataf, table)
    return out.reshape(B, F, D)


# speedup vs baseline: 3.3782x; 1.0016x over previous
import functools

import jax
import jax.numpy as jnp
from jax import lax
from jax.experimental import pallas as pl
from jax.experimental.pallas import tpu as pltpu
from jax.experimental.pallas import tpu_sc as plsc

FIELD_SIZE = 3847
NUM_FIELDS = 26
NC = 2
NS = 16
NW = NC * NS
GROUP = 64
NBUF = 8


def _make_kernel(N, V, D):
    b_per_w = N // NW
    n_groups = b_per_w // GROUP
    mesh = plsc.VectorSubcoreMesh(core_axis_name="c", subcore_axis_name="s")

    @functools.partial(
        pl.kernel,
        mesh=mesh,
        out_type=jax.ShapeDtypeStruct((N, D), jnp.float32),
        scratch_types=[
            pltpu.VMEM((n_groups, GROUP), jnp.int32),
            *[pltpu.VMEM((GROUP, D), jnp.float32) for _ in range(NBUF)],
            *[pltpu.SemaphoreType.DMA for _ in range(NBUF)],
            *[pltpu.SemaphoreType.DMA for _ in range(NBUF)],
        ],
    )
    def embed(data_hbm, table_hbm, out_hbm, idx_v, *rest):
        bufs = rest[:NBUF]
        gsems = rest[NBUF:2 * NBUF]
        ssems = rest[2 * NBUF:]

        wid = lax.axis_index("s") * NC + lax.axis_index("c")
        grp0 = wid * n_groups
        row0 = wid * b_per_w
        lanes = lax.iota(jnp.int32, 16)

        pltpu.sync_copy(data_hbm.at[pl.ds(grp0, n_groups)], idx_v)

        def add_offsets(r):
            p_row = (grp0 + r) * GROUP
            for c in range(GROUP // 16):
                p = p_row + c * 16 + lanes
                off = lax.rem(p, NUM_FIELDS) * FIELD_SIZE
                sl = pl.ds(c * 16, 16)
                idx_v[r, sl] = idx_v[r, sl] + off

        def start_gather(g, b):
            pltpu.async_copy(table_hbm.at[idx_v.at[g]], bufs[b], gsems[b])

        def wait_gather(g, b):
            pltpu.make_async_copy(table_hbm.at[idx_v.at[g]], bufs[b],
                                  gsems[b]).wait()

        def out_slice(g):
            return out_hbm.at[pl.ds(row0 + g * GROUP, GROUP)]

        for b in range(NBUF):
            add_offsets(b)
            start_gather(b, b)

        def step(o, _):
            for b in range(NBUF):
                g = o * NBUF + b
                wait_gather(g, b)
                pltpu.async_copy(bufs[b], out_slice(g), ssems[b])
                gn = g + NBUF

                @pl.when(gn < n_groups)
                def _():
                    pltpu.make_async_copy(bufs[b], out_slice(g),
                                          ssems[b]).wait()
                    add_offsets(gn)
                    start_gather(gn, b)

            return None

        lax.fori_loop(0, n_groups // NBUF, step, None)

        for b in range(NBUF):
            g = n_groups - NBUF + b
            pltpu.make_async_copy(bufs[b], out_slice(g), ssems[b]).wait()

    return embed


def kernel(data, table):
    B, F = data.shape
    V, D = table.shape
    N = B * F
    dataf = data.reshape(N // GROUP, GROUP).astype(jnp.int32)
    embed = _make_kernel(N, V, D)
    out = embed(dataf, table)
    return out.reshape(B, F, D)


# speedup vs baseline: 5.5859x; 1.6535x over previous
import functools

import jax
import jax.numpy as jnp
from jax import lax
from jax.experimental import pallas as pl
from jax.experimental.pallas import tpu as pltpu
from jax.experimental.pallas import tpu_sc as plsc

FIELD_SIZE = 3847
NUM_FIELDS = 26
FPAD = 32
NC = 2
NS = 16
NW = NC * NS
NBG = 8
NBUF = 4


def _make_kernel(B, F, V, D):
    b_per_w = B // NW
    n_groups = b_per_w // NBG
    mesh = plsc.VectorSubcoreMesh(core_axis_name="c", subcore_axis_name="s")

    @functools.partial(
        pl.kernel,
        mesh=mesh,
        out_type=jax.ShapeDtypeStruct((B, F, D), jnp.float32),
        scratch_types=[
            pltpu.VMEM((b_per_w * FPAD,), jnp.int32),
            *[pltpu.VMEM((NBG * F, D), jnp.float32) for _ in range(NBUF)],
            *[pltpu.SemaphoreType.DMA for _ in range(NBUF)],
            *[pltpu.SemaphoreType.DMA for _ in range(NBUF)],
        ],
    )
    def embed(data_hbm, table_hbm, out_hbm, idx_v, *rest):
        bufs = rest[:NBUF]
        gsems = rest[NBUF:2 * NBUF]
        ssems = rest[2 * NBUF:]

        wid = lax.axis_index("s") * NC + lax.axis_index("c")
        b0 = wid * b_per_w
        lanes = lax.iota(jnp.int32, 16)
        off_lo = lanes * FIELD_SIZE
        off_hi = (lanes + 16) * FIELD_SIZE

        pltpu.sync_copy(data_hbm.at[pl.ds(b0 * FPAD, b_per_w * FPAD)], idx_v)

        def add_offsets(r):
            lo = pl.ds(r * FPAD, 16)
            hi = pl.ds(r * FPAD + 16, 16)
            idx_v[lo] = idx_v[lo] + off_lo
            idx_v[hi] = idx_v[hi] + off_hi

        def start_gather(g, b):
            for i in range(NBG):
                r = g * NBG + i
                add_offsets(r)
                pltpu.async_copy(table_hbm.at[idx_v.at[pl.ds(r * FPAD, F)]],
                                 bufs[b].at[pl.ds(i * F, F)], gsems[b])

        def wait_gather(g, b):
            for i in range(NBG):
                r = g * NBG + i
                pltpu.make_async_copy(
                    table_hbm.at[idx_v.at[pl.ds(r * FPAD, F)]],
                    bufs[b].at[pl.ds(i * F, F)], gsems[b]).wait()

        def start_scatter(g, b):
            for i in range(NBG):
                pltpu.async_copy(bufs[b].at[pl.ds(i * F, F)],
                                 out_hbm.at[b0 + g * NBG + i], ssems[b])

        def wait_scatter(g, b):
            for i in range(NBG):
                pltpu.make_async_copy(bufs[b].at[pl.ds(i * F, F)],
                                      out_hbm.at[b0 + g * NBG + i],
                                      ssems[b]).wait()

        for b in range(NBUF):
            start_gather(b, b)

        def step(o, _):
            for b in range(NBUF):
                g = o * NBUF + b
                wait_gather(g, b)
                start_scatter(g, b)
                gn = g + NBUF

                @pl.when(gn < n_groups)
                def _():
                    wait_scatter(g, b)
                    start_gather(gn, b)

            return None

        lax.fori_loop(0, n_groups // NBUF, step, None)

        for b in range(NBUF):
            wait_scatter(n_groups - NBUF + b, b)

    return embed


def kernel(data, table):
    B, F = data.shape
    V, D = table.shape
    datap = jnp.pad(data.astype(jnp.int32), ((0, 0), (0, FPAD - F)))
    embed = _make_kernel(B, F, V, D)
    return embed(datap.reshape(B * FPAD), table)
